# baseline (speedup 1.0000x reference)
import functools

import jax
import jax.numpy as jnp
from jax import lax
from jax.experimental import pallas as pl
from jax.experimental.pallas import tpu as pltpu
from jax.experimental.pallas import tpu_sc as plsc

_NC = 2
_NS = 16
_L = 16

_N_NODES = 50000
_DH = 32
_BUF = 256
_NBUF = 198
_EPT = _BUF * _NBUF
_RPT = _EPT // 128
_NNZ_PAD = _NS * _EPT + 2 * _BUF
_STRIPE = 3128
_N_PAD = _STRIPE * _NS

_BLK = 2000


def _spmm_body(egof, colsh, rowsh, valsh, out,
               acc, colv0, colv1, valv0, valv1, gbuf0, gbuf1, sbuf0, sbuf1,
               rowv0, rowv1, rowv2,
               gsem0, gsem1, ssem0, ssem1, lsem0, lsem1):
    c = lax.axis_index("c")
    s = lax.axis_index("s")
    colv = (colv0, colv1)
    valv = (valv0, valv1)
    gbuf = (gbuf0, gbuf1)
    sbuf = (sbuf0, sbuf1)
    rowv = (rowv0, rowv1, rowv2)
    gsem = (gsem0, gsem1)
    ssem = (ssem0, ssem1)
    lsem = (lsem0, lsem1)

    zero16 = jnp.zeros((_L,), jnp.float32)

    @plsc.parallel_loop(0, _BUF)
    def _zrow(i):
        sbuf0[i, pl.ds(0, _L)] = zero16
        sbuf0[i, pl.ds(_L, _L)] = zero16

    base = s * _STRIPE
    for z in range(_STRIPE // _BUF):
        pltpu.sync_copy(sbuf0, acc.at[pl.ds(base + z * _BUF, _BUF)])
    _zrem = _STRIPE % _BUF
    if _zrem:
        pltpu.sync_copy(sbuf0.at[pl.ds(0, _zrem)],
                        acc.at[pl.ds(base + _STRIPE - _zrem, _zrem)])
    plsc.subcore_barrier()

    col_off = jnp.full((_L,), c * _N_NODES, jnp.int32)
    tbase = s * _RPT

    def fire_copies(b, p, q):
        e0 = tbase * 128 + b * _BUF
        pltpu.async_copy(colsh.at[pl.ds(e0, _BUF)], colv[p], lsem[p])
        pltpu.async_copy(rowsh.at[pl.ds(e0, _BUF)], rowv[q], lsem[p])
        pltpu.async_copy(valsh.at[pl.ds(e0, _BUF)], valv[p], lsem[p])

    def wait_copies(p, q):
        pltpu.make_async_copy(colsh.at[pl.ds(0, _BUF)], colv[p], lsem[p]).wait()
        pltpu.make_async_copy(rowsh.at[pl.ds(0, _BUF)], rowv[q], lsem[p]).wait()
        pltpu.make_async_copy(valsh.at[pl.ds(0, _BUF)], valv[p], lsem[p]).wait()

    def adjust(p):
        for j in range(_BUF // _L):
            colv[p][pl.ds(j * _L, _L)] = colv[p][pl.ds(j * _L, _L)] + col_off

    def fire_gathers(p):
        pltpu.async_copy(egof.at[colv[p]], gbuf[p], gsem[p])

    def wait_gathers(p):
        pltpu.make_async_copy(egof.at[colv[p]], gbuf[p], gsem[p]).wait()

    def fire_scatters(p, q):
        pltpu.async_copy(sbuf[p], acc.at[rowv[q]], ssem[p], add=True)

    def wait_scatters(p, q):
        pltpu.make_async_copy(sbuf[p], acc.at[rowv[q]], ssem[p]).wait()

    def scale(p):
        g = gbuf[p]
        sb = sbuf[p]
        vv = valv[p]

        @plsc.parallel_loop(0, _BUF, unroll=8)
        def _scale(e):
            idx = jnp.full((_L,), e, jnp.int32)
            v = plsc.load_gather(vv, [idx])
            row = g[e, pl.ds(0, 2 * _L)]
            a, b = plsc.unpack(row, format=plsc.PackFormat.INTERLEAVED)
            sb[e, pl.ds(0, _L)] = a * v
            sb[e, pl.ds(_L, _L)] = b * v

    fire_copies(0, 0, 0)
    wait_copies(0, 0)
    adjust(0)
    fire_gathers(0)
    fire_copies(1, 1, 1)

    def _six(i, carry):
        b6 = i * 6
        for h in range(6):
            p = h % 2
            pn = 1 - p
            q = h % 3
            wait_gathers(p)
            wait_copies(pn, (q + 1) % 3)
            adjust(pn)
            if h == 0:
                @pl.when(i > 0)
                def _():
                    wait_scatters(pn, (q + 2) % 3)
            else:
                wait_scatters(pn, (q + 2) % 3)
            fire_gathers(pn)
            scale(p)
            fire_scatters(p, q)
            fire_copies(b6 + h + 2, p, (q + 2) % 3)
        return carry

    lax.fori_loop(0, _NBUF // 6, _six, 0)

    wait_gathers(0)
    wait_copies(1, 1)
    wait_scatters(1, 2)

    plsc.subcore_barrier()
    pltpu.sync_copy(acc.at[pl.ds(base, _STRIPE)],
                    out.at[c, pl.ds(base, _STRIPE)])


_spmm_call = pl.kernel(
    _spmm_body,
    out_type=jax.ShapeDtypeStruct((_NC, _N_PAD, _DH), jnp.float32),
    mesh=plsc.VectorSubcoreMesh(core_axis_name="c", subcore_axis_name="s"),
    scratch_types=[
        pltpu.VMEM_SHARED((_N_PAD, _DH), jnp.float32),
        pltpu.VMEM((_BUF,), jnp.int32),
        pltpu.VMEM((_BUF,), jnp.int32),
        pltpu.VMEM((_BUF,), jnp.float32),
        pltpu.VMEM((_BUF,), jnp.float32),
        pltpu.VMEM((_BUF, _DH), jnp.bfloat16),
        pltpu.VMEM((_BUF, _DH), jnp.bfloat16),
        pltpu.VMEM((_BUF, _DH), jnp.float32),
        pltpu.VMEM((_BUF, _DH), jnp.float32),
        pltpu.VMEM((_BUF,), jnp.int32),
        pltpu.VMEM((_BUF,), jnp.int32),
        pltpu.VMEM((_BUF,), jnp.int32),
        pltpu.SemaphoreType.DMA,
        pltpu.SemaphoreType.DMA,
        pltpu.SemaphoreType.DMA,
        pltpu.SemaphoreType.DMA,
        pltpu.SemaphoreType.DMA,
        pltpu.SemaphoreType.DMA,
    ],
    compiler_params=pltpu.CompilerParams(needs_layout_passes=False,
                                         use_tc_tiling_on_sc=False),
)


def _dense_body(side_ref, ego_ref, w4_ref, b2_ref,
                ego_out_ref, norm_out_ref, stk_out_ref):
    s0 = side_ref[0]
    s1 = side_ref[1]
    e = ego_ref[...]
    x = jnp.concatenate(
        [s0, s1, e[:, :_DH] * s0, e[:, _DH:] * s1], axis=1)
    out = jnp.dot(x, w4_ref[...], preferred_element_type=jnp.float32) + b2_ref[...]
    ego_out_ref[...] = out
    n = jnp.sqrt(jnp.sum(out * out, axis=1, keepdims=True))
    norm_out_ref[...] = out / jnp.maximum(n, 1e-12)
    for h in range(_NC):
        hh = out[:, h * _DH:(h + 1) * _DH]
        inter = jnp.stack([hh[:, :_L], hh[:, _L:]], axis=2).reshape(
            hh.shape[0], _DH)
        stk_out_ref[h] = inter.astype(jnp.bfloat16)


def _dense_layer(side_stk, ego, wgT, bg, wbT, bb):
    n = _N_NODES
    grid = n // _BLK
    w4 = jnp.concatenate([wgT, wbT], axis=0)
    b2 = bg + bb
    return pl.pallas_call(
        _dense_body,
        grid=(grid,),
        in_specs=[
            pl.BlockSpec((_NC, _BLK, _DH), lambda i: (0, i, 0)),
            pl.BlockSpec((_BLK, 64), lambda i: (i, 0)),
            pl.BlockSpec((128, 64), lambda i: (0, 0)),
            pl.BlockSpec((1, 64), lambda i: (0, 0)),
        ],
        out_specs=[
            pl.BlockSpec((_BLK, 64), lambda i: (i, 0)),
            pl.BlockSpec((_BLK, 64), lambda i: (i, 0)),
            pl.BlockSpec((_NC, _BLK, _DH), lambda i: (0, i, 0)),
        ],
        out_shape=[
            jax.ShapeDtypeStruct((n, 64), jnp.float32),
            jax.ShapeDtypeStruct((n, 64), jnp.float32),
            jax.ShapeDtypeStruct((_NC, n, _DH), jnp.bfloat16),
        ],
    )(side_stk, ego, w4, b2)


_NW = _NC * _NS
_GID = 12288
_GPT = _GID // _NW


def _gather_body(t0, t1, t2, t3, idxh, o0, o1, o2, o3, idxv, buf, sem):
    c = lax.axis_index("c")
    s = lax.axis_index("s")
    wid = s * _NC + c
    pltpu.sync_copy(idxh.at[wid], idxv)
    descs = []
    for t, tab in enumerate((t0, t1, t2, t3)):
        for j in range(3):
            descs.append(pltpu.async_copy(
                tab.at[idxv.at[j]],
                buf.at[pl.ds((t * 3 + j) * 128, 128)], sem))
    for d in descs:
        d.wait()
    for t, o in enumerate((o0, o1, o2, o3)):
        pltpu.sync_copy(buf.at[pl.ds(t * _GPT, _GPT)],
                        o.at[pl.ds(wid * _GPT, _GPT)])


_gather_call = pl.kernel(
    _gather_body,
    out_type=[jax.ShapeDtypeStruct((_GID, 64), jnp.float32)] * 4,
    mesh=plsc.VectorSubcoreMesh(core_axis_name="c", subcore_axis_name="s"),
    scratch_types=[
        pltpu.VMEM((3, 128), jnp.int32),
        pltpu.VMEM((4 * _GPT, 64), jnp.float32),
        pltpu.SemaphoreType.DMA,
    ],
    compiler_params=pltpu.CompilerParams(needs_layout_passes=False,
                                         use_tc_tiling_on_sc=False),
)


def kernel(users, pos_items, neg_items, node_drop_flag, user_emb, item_emb,
           W_gc, b_gc, W_bi, b_bi, adj_indices, adj_values):
    n_users = user_emb.shape[0]

    rows = adj_indices[0].astype(jnp.int32)
    cols = adj_indices[1].astype(jnp.int32)
    nnz = rows.shape[0]
    pad_n = _NNZ_PAD - nnz
    pad_idx = (jnp.arange(pad_n, dtype=jnp.int32) * 37) % _N_NODES
    colsp = jnp.concatenate([cols, pad_idx])
    rowsp = jnp.concatenate([rows, pad_idx])
    valsp = jnp.concatenate([adj_values, jnp.zeros((pad_n,), jnp.float32)])

    ego = jnp.concatenate([user_emb, item_emb], axis=0)

    def _to_table(h):
        return jnp.stack([h[:, :_L], h[:, _L:]], axis=2).reshape(
            h.shape[0], _DH).astype(jnp.bfloat16)

    egof = jnp.concatenate([_to_table(ego[:, :_DH]), _to_table(ego[:, _DH:])],
                           axis=0)

    tables = [ego]
    for k in range(W_gc.shape[0]):
        side_stk = _spmm_call(egof, colsp, rowsp, valsp)
        ego, norm, ego_stk = _dense_layer(side_stk, ego,
                                          W_gc[k].T, b_gc[k][None, :],
                                          W_bi[k].T, b_bi[k][None, :])
        egof = ego_stk.reshape(_NC * _N_NODES, _DH)
        tables.append(norm)

    idx = jnp.concatenate([users.astype(jnp.int32),
                           pos_items.astype(jnp.int32) + n_users,
                           neg_items.astype(jnp.int32) + n_users])
    idx2d = idx.reshape(_NW, 3, 128)
    g0, g1, g2, g3 = _gather_call(tables[0], tables[1], tables[2], tables[3],
                                  idx2d)
    g = jnp.concatenate([g0, g1, g2, g3], axis=1)
    return (g[:4096], g[4096:8192], g[8192:])


# speedup vs baseline: 2.5380x; 2.5380x over previous
import functools

import jax
import jax.numpy as jnp
from jax import lax
from jax.experimental import pallas as pl
from jax.experimental.pallas import tpu as pltpu
from jax.experimental.pallas import tpu_sc as plsc

_NC = 2
_NS = 16
_L = 16

_N_NODES = 50000
_DH = 32
_BUF = 384
_SUB = _BUF // 128
_NBUF = 132
_EPT = _BUF * _NBUF
_RPT = _EPT // 128
_NNZ_ROWS = _NS * _RPT + 2 * _SUB
_NNZ_PAD = _NNZ_ROWS * 128
_STRIPE = 3128
_N_PAD = _STRIPE * _NS

_BLK = 5000


def _spmm_body(egof, colsh, rowsh, valsh, out,
               acc, colv0, colv1, valv0, valv1, gbuf0, gbuf1,
               rowv0, rowv1, rowv2,
               gsem0, gsem1, ssem0, ssem1, lsem0, lsem1):
    c = lax.axis_index("c")
    s = lax.axis_index("s")
    colv = (colv0, colv1)
    valv = (valv0, valv1)
    gbuf = (gbuf0, gbuf1)
    rowv = (rowv0, rowv1, rowv2)
    gsem = (gsem0, gsem1)
    ssem = (ssem0, ssem1)
    lsem = (lsem0, lsem1)

    zero16 = jnp.zeros((_L,), jnp.float32)

    @plsc.parallel_loop(0, _BUF)
    def _zrow(i):
        gbuf0[i, pl.ds(0, _L)] = zero16
        gbuf0[i, pl.ds(_L, _L)] = zero16

    base = s * _STRIPE
    for z in range(_STRIPE // _BUF):
        pltpu.sync_copy(gbuf0, acc.at[pl.ds(base + z * _BUF, _BUF)])
    _zrem = _STRIPE % _BUF
    if _zrem:
        pltpu.sync_copy(gbuf0.at[pl.ds(0, _zrem)],
                        acc.at[pl.ds(base + _STRIPE - _zrem, _zrem)])
    plsc.subcore_barrier()

    col_off = jnp.full((_L,), c * _N_NODES, jnp.int32)
    tbase = s * _RPT

    def fire_copies(b, p, q):
        row0 = tbase + b * _SUB
        pltpu.async_copy(colsh.at[pl.ds(row0, _SUB)], colv[p], lsem[p])
        pltpu.async_copy(rowsh.at[pl.ds(row0, _SUB)], rowv[q], lsem[p])
        pltpu.async_copy(valsh.at[pl.ds(row0 * 128, _BUF)], valv[p], lsem[p])

    def wait_copies(p, q):
        pltpu.make_async_copy(colsh.at[pl.ds(tbase, _SUB)], colv[p], lsem[p]).wait()
        pltpu.make_async_copy(rowsh.at[pl.ds(tbase, _SUB)], rowv[q], lsem[p]).wait()
        pltpu.make_async_copy(valsh.at[pl.ds(0, _BUF)], valv[p], lsem[p]).wait()

    def adjust(p):
        for j in range(_SUB):
            for kk in range(128 // _L):
                colv[p][j, pl.ds(kk * _L, _L)] = (
                    colv[p][j, pl.ds(kk * _L, _L)] + col_off)

    def fire_gathers(p):
        for j in range(_SUB):
            pltpu.async_copy(egof.at[colv[p].at[j]],
                             gbuf[p].at[pl.ds(j * 128, 128)], gsem[p])

    def wait_gathers(p):
        for j in range(_SUB):
            pltpu.make_async_copy(egof.at[colv[p].at[j]],
                                  gbuf[p].at[pl.ds(j * 128, 128)],
                                  gsem[p]).wait()

    def fire_scatters(p, q):
        for j in range(_SUB):
            pltpu.async_copy(gbuf[p].at[pl.ds(j * 128, 128)],
                             acc.at[rowv[q].at[j]], ssem[p], add=True)

    def wait_scatters(p, q):
        for j in range(_SUB):
            pltpu.make_async_copy(gbuf[p].at[pl.ds(j * 128, 128)],
                                  acc.at[rowv[q].at[j]], ssem[p]).wait()

    def scale(p):
        g = gbuf[p]
        vv = valv[p]

        @plsc.parallel_loop(0, _BUF, unroll=8)
        def _scale(e):
            idx = jnp.full((_L,), e, jnp.int32)
            v = plsc.load_gather(vv, [idx])
            g[e, pl.ds(0, _L)] = g[e, pl.ds(0, _L)] * v
            g[e, pl.ds(_L, _L)] = g[e, pl.ds(_L, _L)] * v

    fire_copies(0, 0, 0)
    wait_copies(0, 0)
    adjust(0)
    fire_gathers(0)
    fire_copies(1, 1, 1)

    def _six(i, carry):
        b6 = i * 6
        for h in range(6):
            p = h % 2
            pn = 1 - p
            q = h % 3
            wait_gathers(p)
            wait_copies(pn, (q + 1) % 3)
            adjust(pn)
            if h == 0:
                @pl.when(i > 0)
                def _():
                    wait_scatters(pn, (q + 2) % 3)
            else:
                wait_scatters(pn, (q + 2) % 3)
            fire_gathers(pn)
            scale(p)
            fire_scatters(p, q)
            fire_copies(b6 + h + 2, p, (q + 2) % 3)
        return carry

    lax.fori_loop(0, _NBUF // 6, _six, 0)

    wait_gathers(0)
    wait_copies(1, 1)
    wait_scatters(1, 2)

    plsc.subcore_barrier()
    pltpu.sync_copy(acc.at[pl.ds(base, _STRIPE)],
                    out.at[c, pl.ds(base, _STRIPE)])


_spmm_call = pl.kernel(
    _spmm_body,
    out_type=jax.ShapeDtypeStruct((_NC, _N_PAD, _DH), jnp.float32),
    mesh=plsc.VectorSubcoreMesh(core_axis_name="c", subcore_axis_name="s"),
    scratch_types=[
        pltpu.VMEM_SHARED((_N_PAD, _DH), jnp.float32),
        pltpu.VMEM((_SUB, 128), jnp.int32),
        pltpu.VMEM((_SUB, 128), jnp.int32),
        pltpu.VMEM((_BUF,), jnp.float32),
        pltpu.VMEM((_BUF,), jnp.float32),
        pltpu.VMEM((_BUF, _DH), jnp.float32),
        pltpu.VMEM((_BUF, _DH), jnp.float32),
        pltpu.VMEM((_SUB, 128), jnp.int32),
        pltpu.VMEM((_SUB, 128), jnp.int32),
        pltpu.VMEM((_SUB, 128), jnp.int32),
        pltpu.SemaphoreType.DMA,
        pltpu.SemaphoreType.DMA,
        pltpu.SemaphoreType.DMA,
        pltpu.SemaphoreType.DMA,
        pltpu.SemaphoreType.DMA,
        pltpu.SemaphoreType.DMA,
    ],
    compiler_params=pltpu.CompilerParams(needs_layout_passes=False,
                                         use_tc_tiling_on_sc=False),
)


def _dense_body(side_ref, ego_ref, w4_ref, b2_ref,
                ego_out_ref, norm_out_ref, stk_out_ref):
    s0 = side_ref[0]
    s1 = side_ref[1]
    e = ego_ref[...]
    x = jnp.concatenate(
        [s0, s1, e[:, :_DH] * s0, e[:, _DH:] * s1], axis=1)
    out = jnp.dot(x, w4_ref[...], preferred_element_type=jnp.float32) + b2_ref[...]
    ego_out_ref[...] = out
    n = jnp.sqrt(jnp.sum(out * out, axis=1, keepdims=True))
    norm_out_ref[...] = out / jnp.maximum(n, 1e-12)
    stk_out_ref[0] = out[:, :_DH]
    stk_out_ref[1] = out[:, _DH:]


def _dense_layer(side_stk, ego, wgT, bg, wbT, bb):
    n = _N_NODES
    grid = n // _BLK
    w4 = jnp.concatenate([wgT, wbT], axis=0)
    b2 = bg + bb
    return pl.pallas_call(
        _dense_body,
        grid=(grid,),
        in_specs=[
            pl.BlockSpec((_NC, _BLK, _DH), lambda i: (0, i, 0)),
            pl.BlockSpec((_BLK, 64), lambda i: (i, 0)),
            pl.BlockSpec((128, 64), lambda i: (0, 0)),
            pl.BlockSpec((1, 64), lambda i: (0, 0)),
        ],
        out_specs=[
            pl.BlockSpec((_BLK, 64), lambda i: (i, 0)),
            pl.BlockSpec((_BLK, 64), lambda i: (i, 0)),
            pl.BlockSpec((_NC, _BLK, _DH), lambda i: (0, i, 0)),
        ],
        out_shape=[
            jax.ShapeDtypeStruct((n, 64), jnp.float32),
            jax.ShapeDtypeStruct((n, 64), jnp.float32),
            jax.ShapeDtypeStruct((_NC, n, _DH), jnp.float32),
        ],
    )(side_stk, ego, w4, b2)


_NW = _NC * _NS
_GID = 12288
_GPT = _GID // _NW


def _gather_body(t0, t1, t2, t3, idxh, o0, o1, o2, o3, idxv, buf, sem):
    c = lax.axis_index("c")
    s = lax.axis_index("s")
    wid = s * _NC + c
    pltpu.sync_copy(idxh.at[wid], idxv)
    descs = []
    for t, tab in enumerate((t0, t1, t2, t3)):
        for j in range(3):
            descs.append(pltpu.async_copy(
                tab.at[idxv.at[j]],
                buf.at[pl.ds((t * 3 + j) * 128, 128)], sem))
    for d in descs:
        d.wait()
    for t, o in enumerate((o0, o1, o2, o3)):
        pltpu.sync_copy(buf.at[pl.ds(t * _GPT, _GPT)],
                        o.at[pl.ds(wid * _GPT, _GPT)])


_gather_call = pl.kernel(
    _gather_body,
    out_type=[jax.ShapeDtypeStruct((_GID, 64), jnp.float32)] * 4,
    mesh=plsc.VectorSubcoreMesh(core_axis_name="c", subcore_axis_name="s"),
    scratch_types=[
        pltpu.VMEM((3, 128), jnp.int32),
        pltpu.VMEM((4 * _GPT, 64), jnp.float32),
        pltpu.SemaphoreType.DMA,
    ],
    compiler_params=pltpu.CompilerParams(needs_layout_passes=False,
                                         use_tc_tiling_on_sc=False),
)


def kernel(users, pos_items, neg_items, node_drop_flag, user_emb, item_emb,
           W_gc, b_gc, W_bi, b_bi, adj_indices, adj_values):
    n_users = user_emb.shape[0]

    rows = adj_indices[0].astype(jnp.int32)
    cols = adj_indices[1].astype(jnp.int32)
    nnz = rows.shape[0]
    pad_n = _NNZ_PAD - nnz
    pad_idx = jnp.arange(pad_n, dtype=jnp.int32)
    colsp = jnp.concatenate([cols, pad_idx]).reshape(_NNZ_PAD // 128, 128)
    rowsp = jnp.concatenate([rows, pad_idx]).reshape(_NNZ_PAD // 128, 128)
    valsp = jnp.concatenate([adj_values, jnp.zeros((pad_n,), jnp.float32)])

    ego = jnp.concatenate([user_emb, item_emb], axis=0)
    egof = jnp.concatenate([ego[:, :_DH], ego[:, _DH:]], axis=0)

    tables = [ego]
    for k in range(W_gc.shape[0]):
        side_stk = _spmm_call(egof, colsp, rowsp, valsp)
        ego, norm, ego_stk = _dense_layer(side_stk, ego,
                                          W_gc[k].T, b_gc[k][None, :],
                                          W_bi[k].T, b_bi[k][None, :])
        egof = ego_stk.reshape(_NC * _N_NODES, _DH)
        tables.append(norm)

    idx = jnp.concatenate([users.astype(jnp.int32),
                           pos_items.astype(jnp.int32) + n_users,
                           neg_items.astype(jnp.int32) + n_users])
    idx2d = idx.reshape(_NW, 3, 128)
    g0, g1, g2, g3 = _gather_call(tables[0], tables[1], tables[2], tables[3],
                                  idx2d)
    g = jnp.concatenate([g0, g1, g2, g3], axis=1)
    return (g[:4096], g[4096:8192], g[8192:])
